# baseline (speedup 1.0000x reference)
import functools
import jax
import jax.numpy as jnp
from jax import lax
from jax.experimental import pallas as pl
from jax.experimental.pallas import tpu as pltpu

N = 10000
E = 160000
D = 128
NG = 64
DA = 144
RB = 2000
EB = 2000

_f32 = jnp.float32


def _ln(y, g, b):
    mu = jnp.mean(y, axis=-1, keepdims=True)
    var = jnp.mean((y - mu) ** 2, axis=-1, keepdims=True)
    return (y - mu) * lax.rsqrt(var + 1e-5) * g + b


def _k1_body(nf, Wn, bn, P, Q, x_o, u_o, v_o):
    x = jnp.dot(nf[...], Wn[...], preferred_element_type=_f32) + bn[...]
    x_o[...] = x
    u_o[...] = jnp.dot(x, P[...], preferred_element_type=_f32)
    v_o[...] = jnp.dot(x, Q[...], preferred_element_type=_f32)


def _k1(nf, Wn, bn, P, Q):
    grid = N // RB
    return pl.pallas_call(
        _k1_body,
        grid=(grid,),
        in_specs=[
            pl.BlockSpec((RB, 16), lambda i: (i, 0)),
            pl.BlockSpec((16, D), lambda i: (0, 0)),
            pl.BlockSpec((1, D), lambda i: (0, 0)),
            pl.BlockSpec((D, D), lambda i: (0, 0)),
            pl.BlockSpec((D, D), lambda i: (0, 0)),
        ],
        out_specs=[
            pl.BlockSpec((RB, D), lambda i: (i, 0)),
            pl.BlockSpec((RB, D), lambda i: (i, 0)),
            pl.BlockSpec((RB, D), lambda i: (i, 0)),
        ],
        out_shape=[jax.ShapeDtypeStruct((N, D), _f32)] * 3,
        compiler_params=pltpu.CompilerParams(
            dimension_semantics=("arbitrary",)),
    )(nf, Wn, bn, P, Q)


def _k3_body(g, ef, WC, cvec, wtil, c0, m_o, arep_o, mp_o, stat):
    i = pl.program_id(0)
    m = jnp.maximum(
        g[...] + jnp.dot(ef[...], WC[...], preferred_element_type=_f32)
        + cvec[...], 0.0)
    m_o[...] = m
    a = jnp.sum(m * wtil[...], axis=1) + c0[0, 0]
    a = jnp.where(a > 0, a, 0.2 * a)
    arep_o[...] = jnp.broadcast_to(a[:, None], (EB, 16))

    bm = jnp.max(a)
    bs = jnp.sum(jnp.exp(a - bm))

    @pl.when(i == 0)
    def _():
        stat[0] = bm
        stat[1] = bs

    @pl.when(i > 0)
    def _():
        m_old = stat[0]
        s_old = stat[1]
        m_new = jnp.maximum(m_old, bm)
        stat[0] = m_new
        stat[1] = s_old * jnp.exp(m_old - m_new) + bs * jnp.exp(bm - m_new)

    @pl.when(i == pl.num_programs(0) - 1)
    def _():
        mp_o[...] = jnp.reshape(stat[0] + jnp.log(stat[1]), (1, 1))


def _k3(g, ef, WC, cvec, wtil, c0):
    grid = E // EB
    return pl.pallas_call(
        _k3_body,
        grid=(grid,),
        in_specs=[
            pl.BlockSpec((EB, D), lambda i: (i, 0)),
            pl.BlockSpec((EB, 16), lambda i: (i, 0)),
            pl.BlockSpec((16, D), lambda i: (0, 0)),
            pl.BlockSpec((1, D), lambda i: (0, 0)),
            pl.BlockSpec((1, D), lambda i: (0, 0)),
            pl.BlockSpec((1, 1), lambda i: (0, 0), memory_space=pltpu.SMEM),
        ],
        out_specs=[
            pl.BlockSpec((EB, D), lambda i: (i, 0)),
            pl.BlockSpec((EB, 16), lambda i: (i, 0)),
            pl.BlockSpec((1, 1), lambda i: (0, 0)),
        ],
        out_shape=[
            jax.ShapeDtypeStruct((E, D), _f32),
            jax.ShapeDtypeStruct((E, 16), _f32),
            jax.ShapeDtypeStruct((1, 1), _f32),
        ],
        scratch_shapes=[pltpu.SMEM((2,), _f32)],
        compiler_params=pltpu.CompilerParams(
            dimension_semantics=("arbitrary",)),
    )(g, ef, WC, cvec, wtil, c0)


def _k5_body_full(x, agg2, W2, b2, gl, bl, P, Q, x_o, u_o, v_o):
    s = agg2[0, :, :D] + agg2[1, :, :D]
    t = agg2[0, :, D:D + 1] + agg2[1, :, D:D + 1]
    aggf = jnp.dot(s, W2[...], preferred_element_type=_f32) + t * b2[...]
    xn = jnp.maximum(_ln(x[...] + aggf, gl[...], bl[...]), 0.0)
    x_o[...] = xn
    u_o[...] = jnp.dot(xn, P[...], preferred_element_type=_f32)
    v_o[...] = jnp.dot(xn, Q[...], preferred_element_type=_f32)


def _k5_body_last(x, agg2, W2, b2, gl, bl, x_o):
    s = agg2[0, :, :D] + agg2[1, :, :D]
    t = agg2[0, :, D:D + 1] + agg2[1, :, D:D + 1]
    aggf = jnp.dot(s, W2[...], preferred_element_type=_f32) + t * b2[...]
    x_o[...] = jnp.maximum(_ln(x[...] + aggf, gl[...], bl[...]), 0.0)


def _k5(x, agg2, W2, b2, gl, bl, P=None, Q=None):
    grid = N // RB
    last = P is None
    in_specs = [
        pl.BlockSpec((RB, D), lambda i: (i, 0)),
        pl.BlockSpec((2, RB, DA), lambda i: (0, i, 0)),
        pl.BlockSpec((D, D), lambda i: (0, 0)),
        pl.BlockSpec((1, D), lambda i: (0, 0)),
        pl.BlockSpec((1, D), lambda i: (0, 0)),
        pl.BlockSpec((1, D), lambda i: (0, 0)),
    ]
    args = [x, agg2, W2, b2, gl, bl]
    if last:
        body = _k5_body_last
        out_specs = pl.BlockSpec((RB, D), lambda i: (i, 0))
        out_shape = jax.ShapeDtypeStruct((N, D), _f32)
    else:
        body = _k5_body_full
        in_specs += [pl.BlockSpec((D, D), lambda i: (0, 0))] * 2
        args += [P, Q]
        out_specs = [pl.BlockSpec((RB, D), lambda i: (i, 0))] * 3
        out_shape = [jax.ShapeDtypeStruct((N, D), _f32)] * 3
    return pl.pallas_call(
        body,
        grid=(grid,),
        in_specs=in_specs,
        out_specs=out_specs,
        out_shape=out_shape,
        compiler_params=pltpu.CompilerParams(
            dimension_semantics=("arbitrary",)),
    )(*args)


def _k6_body(x, batch3, Wout, bo, gn, bn2, h_o, sums, cnt):
    i = pl.program_id(0)

    @pl.when(i == 0)
    def _():
        sums[...] = jnp.zeros((NG, D), _f32)
        cnt[...] = jnp.zeros((NG, D), _f32)

    ids = batch3[0, 0, :]
    iota = lax.broadcasted_iota(jnp.int32, (RB, NG), 1)
    oh = (ids[:, None] == iota).astype(_f32)
    sums[...] += lax.dot_general(oh, x[...], (((0,), (0,)), ((), ())),
                                 preferred_element_type=_f32)
    cnt[...] += jnp.broadcast_to(jnp.sum(oh, axis=0)[:, None], (NG, D))

    @pl.when(i == pl.num_programs(0) - 1)
    def _():
        pooled = sums[...] / jnp.maximum(cnt[...], 1.0)
        o = jnp.dot(pooled, Wout[...], preferred_element_type=_f32) + bo[...]
        h_o[...] = _ln(o, gn[...], bn2[...])


def _k6(x, batch3, Wout, bo, gn, bn2):
    grid = N // RB
    return pl.pallas_call(
        _k6_body,
        grid=(grid,),
        in_specs=[
            pl.BlockSpec((RB, D), lambda i: (i, 0)),
            pl.BlockSpec((1, 1, RB), lambda i: (i, 0, 0)),
            pl.BlockSpec((D, D), lambda i: (0, 0)),
            pl.BlockSpec((1, D), lambda i: (0, 0)),
            pl.BlockSpec((1, D), lambda i: (0, 0)),
            pl.BlockSpec((1, D), lambda i: (0, 0)),
        ],
        out_specs=pl.BlockSpec((NG, D), lambda i: (0, 0)),
        out_shape=jax.ShapeDtypeStruct((NG, D), _f32),
        scratch_shapes=[pltpu.VMEM((NG, D), _f32), pltpu.VMEM((NG, D), _f32)],
        compiler_params=pltpu.CompilerParams(
            dimension_semantics=("arbitrary",)),
    )(x, batch3, Wout, bo, gn, bn2)


def _k7_body(ha, hb, lw, W1d, W1a, W1b, rb1, rg, rbeta, RW2, rb2,
             HW1, hb1, HW2, hb2, out_o):
    a = ha[...]
    b = hb[...]
    diff = b - a
    w = jnp.exp(lw[...])
    dist = jnp.sqrt(jnp.sum(diff * diff * w, axis=1, keepdims=True) + 1e-8)
    r = (dist * W1d[...]
         + jnp.dot(a, W1a[...], preferred_element_type=_f32)
         + jnp.dot(b, W1b[...], preferred_element_type=_f32) + rb1[...])
    r = jnp.maximum(r, 0.0)
    r = _ln(r, rg[...], rbeta[...])
    r = jnp.maximum(jnp.dot(r, RW2[...], preferred_element_type=_f32)
                    + rb2[...], 0.0)
    h = jnp.maximum(jnp.dot(r, HW1[...], preferred_element_type=_f32)
                    + hb1[...], 0.0)
    out_o[...] = jnp.sum(h * HW2[...], axis=1, keepdims=True) + hb2[...]


def _k7(ha, hb, lw, W1d, W1a, W1b, rb1, rg, rbeta, RW2, rb2, HW1, hb1, HW2, hb2):
    full = lambda s: pl.BlockSpec(s, lambda: tuple(0 for _ in s))
    args = [ha, hb, lw, W1d, W1a, W1b, rb1, rg, rbeta, RW2, rb2, HW1, hb1, HW2, hb2]
    return pl.pallas_call(
        _k7_body,
        in_specs=[full(x.shape) for x in args],
        out_specs=full((NG, 1)),
        out_shape=jax.ShapeDtypeStruct((NG, 1), _f32),
    )(*args)


def _gather(u, v, src, dst):
    return u[dst] + v[src]


def _scatter(m, arep, mp, dst):
    w = jnp.exp(arep[:, 0] - mp[0, 0])
    s = jnp.zeros((N, D), _f32).at[dst].add(m * w[:, None])
    t = jnp.zeros((N,), _f32).at[dst].add(w)
    agg2 = jnp.zeros((2, N, DA), _f32)
    agg2 = agg2.at[0, :, :D].set(s).at[0, :, D].set(t)
    return agg2


def _encode(nf, ei, ef, batch, Wn, bn, We, be, W1s, b1s, W2s, b2s, Was, bas,
            gs, betas, Wout, bo, gn, bn2):
    src = ei[0]
    dst = ei[1]
    Ps, Qs, WCs, cvecs, wtils, c0s = [], [], [], [], [], []
    for l in range(3):
        A, B, C = W1s[l][:D], W1s[l][D:2 * D], W1s[l][2 * D:]
        Ps.append(A - B)
        Qs.append(B)
        WCs.append(We @ C)
        cvecs.append((be @ C + b1s[l]).reshape(1, D))
        wt = (W2s[l] @ Was[l]).reshape(1, D)
        wtils.append(wt)
        c0s.append((b2s[l] @ Was[l] + bas[l]).reshape(1, 1))

    x, u, v = _k1(nf, Wn, bn.reshape(1, D), Ps[0], Qs[0])
    for l in range(3):
        g = _gather(u, v, src, dst)
        m, arep, mp = _k3(g, ef, WCs[l], cvecs[l], wtils[l], c0s[l])
        agg2 = _scatter(m, arep, mp, dst)
        if l < 2:
            x, u, v = _k5(x, agg2, W2s[l], b2s[l].reshape(1, D),
                          gs[l].reshape(1, D), betas[l].reshape(1, D),
                          Ps[l + 1], Qs[l + 1])
        else:
            x = _k5(x, agg2, W2s[l], b2s[l].reshape(1, D),
                    gs[l].reshape(1, D), betas[l].reshape(1, D))
    batch3 = batch.reshape(N // RB, 1, RB)
    return _k6(x, batch3, Wout, bo.reshape(1, D), gn.reshape(1, D),
               bn2.reshape(1, D))


def kernel(node_embed_W, node_embed_b, edge_embed_W, edge_embed_b, conv_W1,
           conv_b1, conv_W2, conv_b2, conv_Wa, conv_ba, conv_g, conv_beta,
           out_proj_W, out_proj_b, out_norm_g, out_norm_b, log_weights,
           res_W1, res_b1, res_g, res_beta, res_W2, res_b2, head_W1, head_b1,
           head_W2, head_b2, node_features_a, edge_index_a, edge_features_a,
           batch_a, node_features_b, edge_index_b, edge_features_b, batch_b):
    enc = functools.partial(
        _encode, Wn=node_embed_W, bn=node_embed_b, We=edge_embed_W,
        be=edge_embed_b, W1s=conv_W1, b1s=conv_b1, W2s=conv_W2, b2s=conv_b2,
        Was=conv_Wa, bas=conv_ba, gs=conv_g, betas=conv_beta, Wout=out_proj_W,
        bo=out_proj_b, gn=out_norm_g, bn2=out_norm_b)
    h_a = enc(node_features_a, edge_index_a, edge_features_a, batch_a)
    h_b = enc(node_features_b, edge_index_b, edge_features_b, batch_b)
    return _k7(h_a, h_b, log_weights.reshape(1, D),
               res_W1[0:1], res_W1[1:D + 1], res_W1[D + 1:2 * D + 1],
               res_b1.reshape(1, D), res_g.reshape(1, D),
               res_beta.reshape(1, D), res_W2, res_b2.reshape(1, D // 2),
               head_W1, head_b1.reshape(1, 32), head_W2.reshape(1, 32),
               head_b2.reshape(1, 1))


# speedup vs baseline: 2.0013x; 2.0013x over previous
import functools
import jax
import jax.numpy as jnp
from jax import lax
from jax.experimental import pallas as pl
from jax.experimental.pallas import tpu as pltpu
from jax.experimental.pallas import tpu_sc as plsc

N = 10000
E = 160000
D = 128
NG = 64
DA = 144
RB = 2000
EB = 2000

_f32 = jnp.float32


def _ln(y, g, b):
    mu = jnp.mean(y, axis=-1, keepdims=True)
    var = jnp.mean((y - mu) ** 2, axis=-1, keepdims=True)
    return (y - mu) * lax.rsqrt(var + 1e-5) * g + b


def _k1_body(nf, Wn, bn, P, Q, x_o, u_o, v_o):
    x = jnp.dot(nf[...], Wn[...], preferred_element_type=_f32) + bn[...]
    x_o[...] = x
    u_o[...] = jnp.dot(x, P[...], preferred_element_type=_f32)
    v_o[...] = jnp.dot(x, Q[...], preferred_element_type=_f32)


def _k1(nf, Wn, bn, P, Q):
    grid = N // RB
    return pl.pallas_call(
        _k1_body,
        grid=(grid,),
        in_specs=[
            pl.BlockSpec((RB, 16), lambda i: (i, 0)),
            pl.BlockSpec((16, D), lambda i: (0, 0)),
            pl.BlockSpec((1, D), lambda i: (0, 0)),
            pl.BlockSpec((D, D), lambda i: (0, 0)),
            pl.BlockSpec((D, D), lambda i: (0, 0)),
        ],
        out_specs=[
            pl.BlockSpec((RB, D), lambda i: (i, 0)),
            pl.BlockSpec((RB, D), lambda i: (i, 0)),
            pl.BlockSpec((RB, D), lambda i: (i, 0)),
        ],
        out_shape=[jax.ShapeDtypeStruct((N, D), _f32)] * 3,
        compiler_params=pltpu.CompilerParams(
            dimension_semantics=("arbitrary",)),
    )(nf, Wn, bn, P, Q)


def _k3_body(gu, gv, ef, WC, cvec, wtil, c0, kap, m_o, arep_o, mp_o, stat):
    i = pl.program_id(0)
    m = jnp.maximum(
        gu[...] + gv[...]
        + jnp.dot(ef[...], WC[...], preferred_element_type=_f32)
        + cvec[...], 0.0)
    m_o[...] = m + kap[...]
    a = jnp.sum(m * wtil[...], axis=1) + c0[0, 0]
    a = jnp.where(a > 0, a, 0.2 * a)
    arep_o[...] = jnp.broadcast_to(a[:, None], (EB, 16))

    bm = jnp.max(a)
    bs = jnp.sum(jnp.exp(a - bm))

    @pl.when(i == 0)
    def _():
        stat[0] = bm
        stat[1] = bs

    @pl.when(i > 0)
    def _():
        m_old = stat[0]
        s_old = stat[1]
        m_new = jnp.maximum(m_old, bm)
        stat[0] = m_new
        stat[1] = s_old * jnp.exp(m_old - m_new) + bs * jnp.exp(bm - m_new)

    @pl.when(i == pl.num_programs(0) - 1)
    def _():
        mp_o[...] = jnp.reshape(stat[0] + jnp.log(stat[1]), (1, 1))


def _k3(gu, gv, ef, WC, cvec, wtil, c0, kap):
    grid = E // EB
    return pl.pallas_call(
        _k3_body,
        grid=(grid,),
        in_specs=[
            pl.BlockSpec((EB, D), lambda i: (i, 0)),
            pl.BlockSpec((EB, D), lambda i: (i, 0)),
            pl.BlockSpec((EB, 16), lambda i: (i, 0)),
            pl.BlockSpec((16, D), lambda i: (0, 0)),
            pl.BlockSpec((1, D), lambda i: (0, 0)),
            pl.BlockSpec((1, D), lambda i: (0, 0)),
            pl.BlockSpec((1, 1), lambda i: (0, 0), memory_space=pltpu.SMEM),
            pl.BlockSpec((1, D), lambda i: (0, 0)),
        ],
        out_specs=[
            pl.BlockSpec((EB, D), lambda i: (i, 0)),
            pl.BlockSpec((EB, 16), lambda i: (i, 0)),
            pl.BlockSpec((1, 1), lambda i: (0, 0)),
        ],
        out_shape=[
            jax.ShapeDtypeStruct((E, D), _f32),
            jax.ShapeDtypeStruct((E, 16), _f32),
            jax.ShapeDtypeStruct((1, 1), _f32),
        ],
        scratch_shapes=[pltpu.SMEM((2,), _f32)],
        compiler_params=pltpu.CompilerParams(
            dimension_semantics=("arbitrary",)),
    )(gu, gv, ef, WC, cvec, wtil, c0, kap)


def _k5_body_full(x, agg2, W2, gl, bl, P, Q, x_o, u_o, v_o):
    s = agg2[0] + agg2[1]
    aggf = jnp.dot(s, W2[...], preferred_element_type=_f32)
    xn = jnp.maximum(_ln(x[...] + aggf, gl[...], bl[...]), 0.0)
    x_o[...] = xn
    u_o[...] = jnp.dot(xn, P[...], preferred_element_type=_f32)
    v_o[...] = jnp.dot(xn, Q[...], preferred_element_type=_f32)


def _k5_body_last(x, agg2, W2, gl, bl, x_o):
    s = agg2[0] + agg2[1]
    aggf = jnp.dot(s, W2[...], preferred_element_type=_f32)
    x_o[...] = jnp.maximum(_ln(x[...] + aggf, gl[...], bl[...]), 0.0)


def _k5(x, agg2, W2, gl, bl, P=None, Q=None):
    grid = N // RB
    last = P is None
    in_specs = [
        pl.BlockSpec((RB, D), lambda i: (i, 0)),
        pl.BlockSpec((2, RB, D), lambda i: (0, i, 0)),
        pl.BlockSpec((D, D), lambda i: (0, 0)),
        pl.BlockSpec((1, D), lambda i: (0, 0)),
        pl.BlockSpec((1, D), lambda i: (0, 0)),
    ]
    args = [x, agg2, W2, gl, bl]
    if last:
        body = _k5_body_last
        out_specs = pl.BlockSpec((RB, D), lambda i: (i, 0))
        out_shape = jax.ShapeDtypeStruct((N, D), _f32)
    else:
        body = _k5_body_full
        in_specs += [pl.BlockSpec((D, D), lambda i: (0, 0))] * 2
        args += [P, Q]
        out_specs = [pl.BlockSpec((RB, D), lambda i: (i, 0))] * 3
        out_shape = [jax.ShapeDtypeStruct((N, D), _f32)] * 3
    return pl.pallas_call(
        body,
        grid=(grid,),
        in_specs=in_specs,
        out_specs=out_specs,
        out_shape=out_shape,
        compiler_params=pltpu.CompilerParams(
            dimension_semantics=("arbitrary",)),
    )(*args)


def _k6_body(x, batch3, Wout, bo, gn, bn2, h_o, sums, cnt):
    i = pl.program_id(0)

    @pl.when(i == 0)
    def _():
        sums[...] = jnp.zeros((NG, D), _f32)
        cnt[...] = jnp.zeros((NG, D), _f32)

    ids = batch3[0, 0, :]
    iota = lax.broadcasted_iota(jnp.int32, (RB, NG), 1)
    oh = (ids[:, None] == iota).astype(_f32)
    sums[...] += lax.dot_general(oh, x[...], (((0,), (0,)), ((), ())),
                                 preferred_element_type=_f32)
    cnt[...] += jnp.broadcast_to(jnp.sum(oh, axis=0)[:, None], (NG, D))

    @pl.when(i == pl.num_programs(0) - 1)
    def _():
        pooled = sums[...] / jnp.maximum(cnt[...], 1.0)
        o = jnp.dot(pooled, Wout[...], preferred_element_type=_f32) + bo[...]
        h_o[...] = _ln(o, gn[...], bn2[...])


def _k6(x, batch3, Wout, bo, gn, bn2):
    grid = N // RB
    return pl.pallas_call(
        _k6_body,
        grid=(grid,),
        in_specs=[
            pl.BlockSpec((RB, D), lambda i: (i, 0)),
            pl.BlockSpec((1, 1, RB), lambda i: (i, 0, 0)),
            pl.BlockSpec((D, D), lambda i: (0, 0)),
            pl.BlockSpec((1, D), lambda i: (0, 0)),
            pl.BlockSpec((1, D), lambda i: (0, 0)),
            pl.BlockSpec((1, D), lambda i: (0, 0)),
        ],
        out_specs=pl.BlockSpec((NG, D), lambda i: (0, 0)),
        out_shape=jax.ShapeDtypeStruct((NG, D), _f32),
        scratch_shapes=[pltpu.VMEM((NG, D), _f32), pltpu.VMEM((NG, D), _f32)],
        compiler_params=pltpu.CompilerParams(
            dimension_semantics=("arbitrary",)),
    )(x, batch3, Wout, bo, gn, bn2)


def _k7_body(ha, hb, lw, W1d, W1a, W1b, rb1, rg, rbeta, RW2, rb2,
             HW1, hb1, HW2, hb2, out_o):
    a = ha[...]
    b = hb[...]
    diff = b - a
    w = jnp.exp(lw[...])
    dist = jnp.sqrt(jnp.sum(diff * diff * w, axis=1, keepdims=True) + 1e-8)
    r = (dist * W1d[...]
         + jnp.dot(a, W1a[...], preferred_element_type=_f32)
         + jnp.dot(b, W1b[...], preferred_element_type=_f32) + rb1[...])
    r = jnp.maximum(r, 0.0)
    r = _ln(r, rg[...], rbeta[...])
    r = jnp.maximum(jnp.dot(r, RW2[...], preferred_element_type=_f32)
                    + rb2[...], 0.0)
    h = jnp.maximum(jnp.dot(r, HW1[...], preferred_element_type=_f32)
                    + hb1[...], 0.0)
    out_o[...] = jnp.sum(h * HW2[...], axis=1, keepdims=True) + hb2[...]


def _k7(ha, hb, lw, W1d, W1a, W1b, rb1, rg, rbeta, RW2, rb2, HW1, hb1, HW2, hb2):
    full = lambda s: pl.BlockSpec(s, lambda: tuple(0 for _ in s))
    args = [ha, hb, lw, W1d, W1a, W1b, rb1, rg, rbeta, RW2, rb2, HW1, hb1, HW2, hb2]
    return pl.pallas_call(
        _k7_body,
        in_specs=[full(x.shape) for x in args],
        out_specs=full((NG, 1)),
        out_shape=jax.ShapeDtypeStruct((NG, 1), _f32),
    )(*args)


NC = 2
NS = 16
NW = NC * NS
EPW = E // NW
CH = 200
NCH = EPW // CH
CS = 40
NCS = EPW // CS
NP = 10240
RPS = NP // NS

_sc_mesh = plsc.VectorSubcoreMesh(core_axis_name="c", subcore_axis_name="s")


@functools.partial(
    pl.kernel,
    out_type=[jax.ShapeDtypeStruct((E, D), _f32)] * 2,
    mesh=_sc_mesh,
    scratch_types=[
        pltpu.VMEM((CH,), jnp.int32),
        pltpu.VMEM((CH,), jnp.int32),
        pltpu.VMEM((CH, D), _f32),
        pltpu.VMEM((CH, D), _f32),
        pltpu.SemaphoreType.DMA,
        pltpu.SemaphoreType.DMA,
    ],
)
def _k2_sc(u_hbm, v_hbm, dst_hbm, src_hbm, gu_hbm, gv_hbm,
           idx_d, idx_s, bu, bv, sem1, sem2):
    wid = lax.axis_index("s") * NC + lax.axis_index("c")
    base = wid * EPW

    def chunk(i, carry):
        off = base + i * CH
        pltpu.sync_copy(dst_hbm.at[pl.ds(off, CH)], idx_d)
        pltpu.sync_copy(src_hbm.at[pl.ds(off, CH)], idx_s)
        cu = pltpu.async_copy(u_hbm.at[idx_d], bu, sem1)
        cv = pltpu.async_copy(v_hbm.at[idx_s], bv, sem2)
        cu.wait()
        cv.wait()
        pltpu.sync_copy(bu, gu_hbm.at[pl.ds(off, CH)])
        pltpu.sync_copy(bv, gv_hbm.at[pl.ds(off, CH)])
        return carry

    lax.fori_loop(0, NCH, chunk, 0)


@functools.partial(
    pl.kernel,
    out_type=jax.ShapeDtypeStruct((NC, NP, D), _f32),
    mesh=_sc_mesh,
    scratch_types=[
        pltpu.VMEM_SHARED((NP, D), _f32),
        pltpu.VMEM((CS,), jnp.int32),
        pltpu.VMEM((CS, D), _f32),
        pltpu.VMEM((CS, 16), _f32),
        pltpu.VMEM((CS, D), _f32),
        pltpu.VMEM((16,), _f32),
    ],
)
def _k4_sc(m_hbm, arep_hbm, mp_hbm, dst_hbm, agg_hbm,
           shared, idx, bm, ba, buf, mp_v):
    cid = lax.axis_index("c")
    sid = lax.axis_index("s")
    base = cid * (E // NC) + sid * EPW

    pltpu.sync_copy(mp_hbm, mp_v)

    zeros16 = jnp.zeros((16,), _f32)

    def zrow(r, carry):
        for k in range(D // 16):
            buf[r, pl.ds(k * 16, 16)] = zeros16
        return carry

    lax.fori_loop(0, CS, zrow, 0)
    for o in range(0, RPS, CS):
        pltpu.sync_copy(buf, shared.at[pl.ds(sid * RPS + o, CS)])
    plsc.subcore_barrier()

    def chunk(i, carry):
        off = base + i * CS
        pltpu.sync_copy(dst_hbm.at[pl.ds(off, CS)], idx)
        pltpu.sync_copy(m_hbm.at[pl.ds(off, CS)], bm)
        pltpu.sync_copy(arep_hbm.at[pl.ds(off, CS)], ba)

        def row(r, c2):
            w = jnp.exp(ba[r, pl.ds(0, 16)] - mp_v[...])
            for k in range(D // 16):
                sl = pl.ds(k * 16, 16)
                buf[r, sl] = bm[r, sl] * w
            return c2

        lax.fori_loop(0, CS, row, 0)
        pltpu.sync_copy(buf, shared.at[idx], add=True)
        return carry

    lax.fori_loop(0, NCS, chunk, 0)
    plsc.subcore_barrier()

    for o in range(0, RPS, CS):
        pltpu.sync_copy(shared.at[pl.ds(sid * RPS + o, CS)],
                        agg_hbm.at[cid, pl.ds(sid * RPS + o, CS)])


def _gather(u, v, src, dst):
    return _k2_sc(u, v, dst, src)


def _scatter(m, arep, mp, dst):
    mp16 = jnp.broadcast_to(mp.reshape(1), (16,))
    return _k4_sc(m, arep, mp16, dst)


def _encode(nf, ei, ef, batch, Wn, bn, We, be, W1s, b1s, W2s, b2s, Was, bas,
            gs, betas, Wout, bo, gn, bn2):
    src = ei[0]
    dst = ei[1]
    Ps, Qs, WCs, cvecs, wtils, c0s, kappas = [], [], [], [], [], [], []
    for l in range(3):
        A, B, C = W1s[l][:D], W1s[l][D:2 * D], W1s[l][2 * D:]
        Ps.append(A - B)
        Qs.append(B)
        WCs.append(We @ C)
        cvecs.append((be @ C + b1s[l]).reshape(1, D))
        wt = (W2s[l] @ Was[l]).reshape(1, D)
        wtils.append(wt)
        c0s.append((b2s[l] @ Was[l] + bas[l]).reshape(1, 1))
        kappas.append(jnp.linalg.solve(W2s[l].T, b2s[l]).reshape(1, D))

    x, u, v = _k1(nf, Wn, bn.reshape(1, D), Ps[0], Qs[0])
    for l in range(3):
        gu, gv = _gather(u, v, src, dst)
        m, arep, mp = _k3(gu, gv, ef, WCs[l], cvecs[l], wtils[l], c0s[l],
                          kappas[l])
        agg2 = _scatter(m, arep, mp, dst)
        if l < 2:
            x, u, v = _k5(x, agg2, W2s[l],
                          gs[l].reshape(1, D), betas[l].reshape(1, D),
                          Ps[l + 1], Qs[l + 1])
        else:
            x = _k5(x, agg2, W2s[l],
                    gs[l].reshape(1, D), betas[l].reshape(1, D))
    batch3 = batch.reshape(N // RB, 1, RB)
    return _k6(x, batch3, Wout, bo.reshape(1, D), gn.reshape(1, D),
               bn2.reshape(1, D))


def kernel(node_embed_W, node_embed_b, edge_embed_W, edge_embed_b, conv_W1,
           conv_b1, conv_W2, conv_b2, conv_Wa, conv_ba, conv_g, conv_beta,
           out_proj_W, out_proj_b, out_norm_g, out_norm_b, log_weights,
           res_W1, res_b1, res_g, res_beta, res_W2, res_b2, head_W1, head_b1,
           head_W2, head_b2, node_features_a, edge_index_a, edge_features_a,
           batch_a, node_features_b, edge_index_b, edge_features_b, batch_b):
    enc = functools.partial(
        _encode, Wn=node_embed_W, bn=node_embed_b, We=edge_embed_W,
        be=edge_embed_b, W1s=conv_W1, b1s=conv_b1, W2s=conv_W2, b2s=conv_b2,
        Was=conv_Wa, bas=conv_ba, gs=conv_g, betas=conv_beta, Wout=out_proj_W,
        bo=out_proj_b, gn=out_norm_g, bn2=out_norm_b)
    h_a = enc(node_features_a, edge_index_a, edge_features_a, batch_a)
    h_b = enc(node_features_b, edge_index_b, edge_features_b, batch_b)
    return _k7(h_a, h_b, log_weights.reshape(1, D),
               res_W1[0:1], res_W1[1:D + 1], res_W1[D + 1:2 * D + 1],
               res_b1.reshape(1, D), res_g.reshape(1, D),
               res_beta.reshape(1, D), res_W2, res_b2.reshape(1, D // 2),
               head_W1, head_b1.reshape(1, 32), head_W2.reshape(1, 32),
               head_b2.reshape(1, 1))


# speedup vs baseline: 2.7317x; 1.3650x over previous
import functools
import jax
import jax.numpy as jnp
from jax import lax
from jax.experimental import pallas as pl
from jax.experimental.pallas import tpu as pltpu
from jax.experimental.pallas import tpu_sc as plsc

N = 10000
E = 160000
D = 128
NG = 64
DA = 144
RB = 2000
EB = 2000

_f32 = jnp.float32


def _ln(y, g, b):
    mu = jnp.mean(y, axis=-1, keepdims=True)
    var = jnp.mean((y - mu) ** 2, axis=-1, keepdims=True)
    return (y - mu) * lax.rsqrt(var + 1e-5) * g + b


def _k1_body(nf, Wn, bn, P, Q, x_o, u_o, v_o):
    x = jnp.dot(nf[...], Wn[...], preferred_element_type=_f32) + bn[...]
    x_o[...] = x
    u_o[...] = jnp.dot(x, P[...], preferred_element_type=_f32)
    v_o[...] = jnp.dot(x, Q[...], preferred_element_type=_f32)


def _k1(nf, Wn, bn, P, Q):
    grid = N // RB
    return pl.pallas_call(
        _k1_body,
        grid=(grid,),
        in_specs=[
            pl.BlockSpec((RB, 16), lambda i: (i, 0)),
            pl.BlockSpec((16, D), lambda i: (0, 0)),
            pl.BlockSpec((1, D), lambda i: (0, 0)),
            pl.BlockSpec((D, D), lambda i: (0, 0)),
            pl.BlockSpec((D, D), lambda i: (0, 0)),
        ],
        out_specs=[
            pl.BlockSpec((RB, D), lambda i: (i, 0)),
            pl.BlockSpec((RB, D), lambda i: (i, 0)),
            pl.BlockSpec((RB, D), lambda i: (i, 0)),
        ],
        out_shape=[jax.ShapeDtypeStruct((N, D), _f32)] * 3,
        compiler_params=pltpu.CompilerParams(
            dimension_semantics=("arbitrary",)),
    )(nf, Wn, bn, P, Q)


def _k3_body(gu, gv, ef, WC, cvec, wtil, c0, kap, m_o, arep_o, mp_o, stat):
    i = pl.program_id(0)
    m = jnp.maximum(
        gu[...] + gv[...]
        + jnp.dot(ef[...], WC[...], preferred_element_type=_f32)
        + cvec[...], 0.0)
    m_o[...] = m + kap[...]
    a = jnp.sum(m * wtil[...], axis=1) + c0[0, 0]
    a = jnp.where(a > 0, a, 0.2 * a)
    arep_o[...] = jnp.broadcast_to(a[:, None], (EB, 16))

    bm = jnp.max(a)
    bs = jnp.sum(jnp.exp(a - bm))

    @pl.when(i == 0)
    def _():
        stat[0] = bm
        stat[1] = bs

    @pl.when(i > 0)
    def _():
        m_old = stat[0]
        s_old = stat[1]
        m_new = jnp.maximum(m_old, bm)
        stat[0] = m_new
        stat[1] = s_old * jnp.exp(m_old - m_new) + bs * jnp.exp(bm - m_new)

    @pl.when(i == pl.num_programs(0) - 1)
    def _():
        mp_o[...] = jnp.reshape(stat[0] + jnp.log(stat[1]), (1, 1))


def _k3(gu, gv, ef, WC, cvec, wtil, c0, kap):
    grid = E // EB
    return pl.pallas_call(
        _k3_body,
        grid=(grid,),
        in_specs=[
            pl.BlockSpec((EB, D), lambda i: (i, 0)),
            pl.BlockSpec((EB, D), lambda i: (i, 0)),
            pl.BlockSpec((EB, 16), lambda i: (i, 0)),
            pl.BlockSpec((16, D), lambda i: (0, 0)),
            pl.BlockSpec((1, D), lambda i: (0, 0)),
            pl.BlockSpec((1, D), lambda i: (0, 0)),
            pl.BlockSpec((1, 1), lambda i: (0, 0), memory_space=pltpu.SMEM),
            pl.BlockSpec((1, D), lambda i: (0, 0)),
        ],
        out_specs=[
            pl.BlockSpec((EB, D), lambda i: (i, 0)),
            pl.BlockSpec((EB, 16), lambda i: (i, 0)),
            pl.BlockSpec((1, 1), lambda i: (0, 0)),
        ],
        out_shape=[
            jax.ShapeDtypeStruct((E, D), _f32),
            jax.ShapeDtypeStruct((E, 16), _f32),
            jax.ShapeDtypeStruct((1, 1), _f32),
        ],
        scratch_shapes=[pltpu.SMEM((2,), _f32)],
        compiler_params=pltpu.CompilerParams(
            dimension_semantics=("arbitrary",)),
    )(gu, gv, ef, WC, cvec, wtil, c0, kap)


def _k5_body_full(x, agg2, W2, gl, bl, P, Q, x_o, u_o, v_o):
    s = agg2[0] + agg2[1]
    aggf = jnp.dot(s, W2[...], preferred_element_type=_f32)
    xn = jnp.maximum(_ln(x[...] + aggf, gl[...], bl[...]), 0.0)
    x_o[...] = xn
    u_o[...] = jnp.dot(xn, P[...], preferred_element_type=_f32)
    v_o[...] = jnp.dot(xn, Q[...], preferred_element_type=_f32)


def _k5_body_last(x, agg2, W2, gl, bl, x_o):
    s = agg2[0] + agg2[1]
    aggf = jnp.dot(s, W2[...], preferred_element_type=_f32)
    x_o[...] = jnp.maximum(_ln(x[...] + aggf, gl[...], bl[...]), 0.0)


def _k5(x, agg2, W2, gl, bl, P=None, Q=None):
    grid = N // RB
    last = P is None
    in_specs = [
        pl.BlockSpec((RB, D), lambda i: (i, 0)),
        pl.BlockSpec((2, RB, D), lambda i: (0, i, 0)),
        pl.BlockSpec((D, D), lambda i: (0, 0)),
        pl.BlockSpec((1, D), lambda i: (0, 0)),
        pl.BlockSpec((1, D), lambda i: (0, 0)),
    ]
    args = [x, agg2, W2, gl, bl]
    if last:
        body = _k5_body_last
        out_specs = pl.BlockSpec((RB, D), lambda i: (i, 0))
        out_shape = jax.ShapeDtypeStruct((N, D), _f32)
    else:
        body = _k5_body_full
        in_specs += [pl.BlockSpec((D, D), lambda i: (0, 0))] * 2
        args += [P, Q]
        out_specs = [pl.BlockSpec((RB, D), lambda i: (i, 0))] * 3
        out_shape = [jax.ShapeDtypeStruct((N, D), _f32)] * 3
    return pl.pallas_call(
        body,
        grid=(grid,),
        in_specs=in_specs,
        out_specs=out_specs,
        out_shape=out_shape,
        compiler_params=pltpu.CompilerParams(
            dimension_semantics=("arbitrary",)),
    )(*args)


def _k6_body(x, batch3, Wout, bo, gn, bn2, h_o, sums, cnt):
    i = pl.program_id(0)

    @pl.when(i == 0)
    def _():
        sums[...] = jnp.zeros((NG, D), _f32)
        cnt[...] = jnp.zeros((NG, D), _f32)

    ids = batch3[0, 0, :]
    iota = lax.broadcasted_iota(jnp.int32, (RB, NG), 1)
    oh = (ids[:, None] == iota).astype(_f32)
    sums[...] += lax.dot_general(oh, x[...], (((0,), (0,)), ((), ())),
                                 preferred_element_type=_f32)
    cnt[...] += jnp.broadcast_to(jnp.sum(oh, axis=0)[:, None], (NG, D))

    @pl.when(i == pl.num_programs(0) - 1)
    def _():
        pooled = sums[...] / jnp.maximum(cnt[...], 1.0)
        o = jnp.dot(pooled, Wout[...], preferred_element_type=_f32) + bo[...]
        h_o[...] = _ln(o, gn[...], bn2[...])


def _k6(x, batch3, Wout, bo, gn, bn2):
    grid = N // RB
    return pl.pallas_call(
        _k6_body,
        grid=(grid,),
        in_specs=[
            pl.BlockSpec((RB, D), lambda i: (i, 0)),
            pl.BlockSpec((1, 1, RB), lambda i: (i, 0, 0)),
            pl.BlockSpec((D, D), lambda i: (0, 0)),
            pl.BlockSpec((1, D), lambda i: (0, 0)),
            pl.BlockSpec((1, D), lambda i: (0, 0)),
            pl.BlockSpec((1, D), lambda i: (0, 0)),
        ],
        out_specs=pl.BlockSpec((NG, D), lambda i: (0, 0)),
        out_shape=jax.ShapeDtypeStruct((NG, D), _f32),
        scratch_shapes=[pltpu.VMEM((NG, D), _f32), pltpu.VMEM((NG, D), _f32)],
        compiler_params=pltpu.CompilerParams(
            dimension_semantics=("arbitrary",)),
    )(x, batch3, Wout, bo, gn, bn2)


def _k7_body(ha, hb, lw, W1d, W1a, W1b, rb1, rg, rbeta, RW2, rb2,
             HW1, hb1, HW2, hb2, out_o):
    a = ha[...]
    b = hb[...]
    diff = b - a
    w = jnp.exp(lw[...])
    dist = jnp.sqrt(jnp.sum(diff * diff * w, axis=1, keepdims=True) + 1e-8)
    r = (dist * W1d[...]
         + jnp.dot(a, W1a[...], preferred_element_type=_f32)
         + jnp.dot(b, W1b[...], preferred_element_type=_f32) + rb1[...])
    r = jnp.maximum(r, 0.0)
    r = _ln(r, rg[...], rbeta[...])
    r = jnp.maximum(jnp.dot(r, RW2[...], preferred_element_type=_f32)
                    + rb2[...], 0.0)
    h = jnp.maximum(jnp.dot(r, HW1[...], preferred_element_type=_f32)
                    + hb1[...], 0.0)
    out_o[...] = jnp.sum(h * HW2[...], axis=1, keepdims=True) + hb2[...]


def _k7(ha, hb, lw, W1d, W1a, W1b, rb1, rg, rbeta, RW2, rb2, HW1, hb1, HW2, hb2):
    full = lambda s: pl.BlockSpec(s, lambda: tuple(0 for _ in s))
    args = [ha, hb, lw, W1d, W1a, W1b, rb1, rg, rbeta, RW2, rb2, HW1, hb1, HW2, hb2]
    return pl.pallas_call(
        _k7_body,
        in_specs=[full(x.shape) for x in args],
        out_specs=full((NG, 1)),
        out_shape=jax.ShapeDtypeStruct((NG, 1), _f32),
    )(*args)


NC = 2
NS = 16
NW = NC * NS
EPW = E // NW
CH = 400
NCH = EPW // CH
CS = 160
NCS = EPW // CS
NP = 10240
RPS = NP // NS

_sc_mesh = plsc.VectorSubcoreMesh(core_axis_name="c", subcore_axis_name="s")


@functools.partial(
    pl.kernel,
    out_type=[jax.ShapeDtypeStruct((E, D), _f32)] * 2,
    mesh=_sc_mesh,
    scratch_types=[
        pltpu.VMEM((CH,), jnp.int32),
        pltpu.VMEM((CH,), jnp.int32),
        pltpu.VMEM((CH, D), _f32),
        pltpu.VMEM((CH, D), _f32),
        pltpu.SemaphoreType.DMA,
        pltpu.SemaphoreType.DMA,
    ],
)
def _k2_sc(u_hbm, v_hbm, dst_hbm, src_hbm, gu_hbm, gv_hbm,
           idx_d, idx_s, bu, bv, sem1, sem2):
    wid = lax.axis_index("s") * NC + lax.axis_index("c")
    base = wid * EPW

    def chunk(i, carry):
        off = base + i * CH
        pltpu.sync_copy(dst_hbm.at[pl.ds(off, CH)], idx_d)
        pltpu.sync_copy(src_hbm.at[pl.ds(off, CH)], idx_s)
        cu = pltpu.async_copy(u_hbm.at[idx_d], bu, sem1)
        cv = pltpu.async_copy(v_hbm.at[idx_s], bv, sem2)
        cu.wait()
        cv.wait()
        pltpu.sync_copy(bu, gu_hbm.at[pl.ds(off, CH)])
        pltpu.sync_copy(bv, gv_hbm.at[pl.ds(off, CH)])
        return carry

    lax.fori_loop(0, NCH, chunk, 0)

    rem = EPW - NCH * CH
    off = base + NCH * CH
    pltpu.sync_copy(dst_hbm.at[pl.ds(off, rem)], idx_d.at[pl.ds(0, rem)])
    pltpu.sync_copy(src_hbm.at[pl.ds(off, rem)], idx_s.at[pl.ds(0, rem)])
    cu = pltpu.async_copy(u_hbm.at[idx_d.at[pl.ds(0, rem)]],
                          bu.at[pl.ds(0, rem)], sem1)
    cv = pltpu.async_copy(v_hbm.at[idx_s.at[pl.ds(0, rem)]],
                          bv.at[pl.ds(0, rem)], sem2)
    cu.wait()
    cv.wait()
    pltpu.sync_copy(bu.at[pl.ds(0, rem)], gu_hbm.at[pl.ds(off, rem)])
    pltpu.sync_copy(bv.at[pl.ds(0, rem)], gv_hbm.at[pl.ds(off, rem)])


@functools.partial(
    pl.kernel,
    out_type=jax.ShapeDtypeStruct((NC, NP, D), _f32),
    mesh=_sc_mesh,
    scratch_types=[
        pltpu.VMEM_SHARED((NP, D), _f32),
        pltpu.VMEM((CS,), jnp.int32),
        pltpu.VMEM((EPW - (EPW // CS) * CS,), jnp.int32),
        pltpu.VMEM((CS, D), _f32),
        pltpu.VMEM((CS, 16), _f32),
        pltpu.VMEM((16,), _f32),
    ],
)
def _k4_sc(m_hbm, arep_hbm, mp_hbm, dst_hbm, agg_hbm,
           shared, idx, idx2, bm, ba, mp_v):
    cid = lax.axis_index("c")
    sid = lax.axis_index("s")
    base = cid * (E // NC) + sid * EPW

    pltpu.sync_copy(mp_hbm, mp_v)

    zeros16 = jnp.zeros((16,), _f32)

    def zrow(r, carry):
        for k in range(D // 16):
            bm[r, pl.ds(k * 16, 16)] = zeros16
        return carry

    lax.fori_loop(0, CS, zrow, 0)
    for o in range(0, RPS, CS):
        sz = min(CS, RPS - o)
        pltpu.sync_copy(bm.at[pl.ds(0, sz)],
                        shared.at[pl.ds(sid * RPS + o, sz)])
    plsc.subcore_barrier()

    def do_chunk(off, sz, idxr):
        pltpu.sync_copy(dst_hbm.at[pl.ds(off, sz)], idxr)
        bms = bm if sz == CS else bm.at[pl.ds(0, sz)]
        pltpu.sync_copy(m_hbm.at[pl.ds(off, sz)], bms)
        pltpu.sync_copy(arep_hbm.at[pl.ds(off, sz)], ba.at[pl.ds(0, sz)])

        def row(r, c2):
            w = jnp.exp(ba[r, pl.ds(0, 16)] - mp_v[...])
            for k in range(D // 16):
                sl = pl.ds(k * 16, 16)
                bm[r, sl] = bm[r, sl] * w
            return c2

        lax.fori_loop(0, sz, row, 0)
        pltpu.sync_copy(bms, shared.at[idxr], add=True)

    def chunk(i, carry):
        do_chunk(base + i * CS, CS, idx)
        return carry

    lax.fori_loop(0, NCS, chunk, 0)
    do_chunk(base + NCS * CS, EPW - NCS * CS, idx2)
    plsc.subcore_barrier()

    pltpu.sync_copy(shared.at[pl.ds(sid * RPS, RPS)],
                    agg_hbm.at[cid, pl.ds(sid * RPS, RPS)])


def _gather(u, v, src, dst):
    return _k2_sc(u, v, dst, src)


def _scatter(m, arep, mp, dst):
    mp16 = jnp.broadcast_to(mp.reshape(1), (16,))
    return _k4_sc(m, arep, mp16, dst)


def _encode(nf, ei, ef, batch, Wn, bn, We, be, W1s, b1s, W2s, b2s, Was, bas,
            gs, betas, Wout, bo, gn, bn2):
    src = ei[0]
    dst = ei[1]
    Ps, Qs, WCs, cvecs, wtils, c0s, kappas = [], [], [], [], [], [], []
    for l in range(3):
        A, B, C = W1s[l][:D], W1s[l][D:2 * D], W1s[l][2 * D:]
        Ps.append(A - B)
        Qs.append(B)
        WCs.append(We @ C)
        cvecs.append((be @ C + b1s[l]).reshape(1, D))
        wt = (W2s[l] @ Was[l]).reshape(1, D)
        wtils.append(wt)
        c0s.append((b2s[l] @ Was[l] + bas[l]).reshape(1, 1))
        kappas.append(jnp.linalg.solve(W2s[l].T, b2s[l]).reshape(1, D))

    x, u, v = _k1(nf, Wn, bn.reshape(1, D), Ps[0], Qs[0])
    for l in range(3):
        gu, gv = _gather(u, v, src, dst)
        m, arep, mp = _k3(gu, gv, ef, WCs[l], cvecs[l], wtils[l], c0s[l],
                          kappas[l])
        agg2 = _scatter(m, arep, mp, dst)
        if l < 2:
            x, u, v = _k5(x, agg2, W2s[l],
                          gs[l].reshape(1, D), betas[l].reshape(1, D),
                          Ps[l + 1], Qs[l + 1])
        else:
            x = _k5(x, agg2, W2s[l],
                    gs[l].reshape(1, D), betas[l].reshape(1, D))
    batch3 = batch.reshape(N // RB, 1, RB)
    return _k6(x, batch3, Wout, bo.reshape(1, D), gn.reshape(1, D),
               bn2.reshape(1, D))


def kernel(node_embed_W, node_embed_b, edge_embed_W, edge_embed_b, conv_W1,
           conv_b1, conv_W2, conv_b2, conv_Wa, conv_ba, conv_g, conv_beta,
           out_proj_W, out_proj_b, out_norm_g, out_norm_b, log_weights,
           res_W1, res_b1, res_g, res_beta, res_W2, res_b2, head_W1, head_b1,
           head_W2, head_b2, node_features_a, edge_index_a, edge_features_a,
           batch_a, node_features_b, edge_index_b, edge_features_b, batch_b):
    enc = functools.partial(
        _encode, Wn=node_embed_W, bn=node_embed_b, We=edge_embed_W,
        be=edge_embed_b, W1s=conv_W1, b1s=conv_b1, W2s=conv_W2, b2s=conv_b2,
        Was=conv_Wa, bas=conv_ba, gs=conv_g, betas=conv_beta, Wout=out_proj_W,
        bo=out_proj_b, gn=out_norm_g, bn2=out_norm_b)
    h_a = enc(node_features_a, edge_index_a, edge_features_a, batch_a)
    h_b = enc(node_features_b, edge_index_b, edge_features_b, batch_b)
    return _k7(h_a, h_b, log_weights.reshape(1, D),
               res_W1[0:1], res_W1[1:D + 1], res_W1[D + 1:2 * D + 1],
               res_b1.reshape(1, D), res_g.reshape(1, D),
               res_beta.reshape(1, D), res_W2, res_b2.reshape(1, D // 2),
               head_W1, head_b1.reshape(1, 32), head_W2.reshape(1, 32),
               head_b2.reshape(1, 1))


# speedup vs baseline: 2.7740x; 1.0155x over previous
import functools
import jax
import jax.numpy as jnp
from jax import lax
from jax.experimental import pallas as pl
from jax.experimental.pallas import tpu as pltpu
from jax.experimental.pallas import tpu_sc as plsc

N = 10000
E = 160000
D = 128
NG = 64
DA = 144
RB = 2000
EB = 2000

_f32 = jnp.float32


def _ln(y, g, b):
    mu = jnp.mean(y, axis=-1, keepdims=True)
    var = jnp.mean((y - mu) ** 2, axis=-1, keepdims=True)
    return (y - mu) * lax.rsqrt(var + 1e-5) * g + b


def _k1_body(nf, Wn, bn, P, Q, x_o, u_o, v_o):
    x = jnp.dot(nf[...], Wn[...], preferred_element_type=_f32) + bn[...]
    x_o[...] = x
    u_o[...] = jnp.dot(x, P[...], preferred_element_type=_f32)
    v_o[...] = jnp.dot(x, Q[...], preferred_element_type=_f32)


def _k1(nf, Wn, bn, P, Q):
    grid = N // RB
    return pl.pallas_call(
        _k1_body,
        grid=(grid,),
        in_specs=[
            pl.BlockSpec((RB, 16), lambda i: (i, 0)),
            pl.BlockSpec((16, D), lambda i: (0, 0)),
            pl.BlockSpec((1, D), lambda i: (0, 0)),
            pl.BlockSpec((D, D), lambda i: (0, 0)),
            pl.BlockSpec((D, D), lambda i: (0, 0)),
        ],
        out_specs=[
            pl.BlockSpec((RB, D), lambda i: (i, 0)),
            pl.BlockSpec((RB, D), lambda i: (i, 0)),
            pl.BlockSpec((RB, D), lambda i: (i, 0)),
        ],
        out_shape=[jax.ShapeDtypeStruct((N, D), _f32)] * 3,
        compiler_params=pltpu.CompilerParams(
            dimension_semantics=("arbitrary",)),
    )(nf, Wn, bn, P, Q)


def _k3_body(gu, gv, ef, WC, cvec, wtil, c0, kap, m_o, arep_o, mp_o, stat):
    i = pl.program_id(0)
    m = jnp.maximum(
        gu[...] + gv[...]
        + jnp.dot(ef[...], WC[...], preferred_element_type=_f32)
        + cvec[...], 0.0)
    m_o[...] = m + kap[...]
    a = jnp.sum(m * wtil[...], axis=1) + c0[0, 0]
    a = jnp.where(a > 0, a, 0.2 * a)
    arep_o[...] = jnp.broadcast_to(a[:, None], (EB, 16))

    bm = jnp.max(a)
    bs = jnp.sum(jnp.exp(a - bm))

    @pl.when(i == 0)
    def _():
        stat[0] = bm
        stat[1] = bs

    @pl.when(i > 0)
    def _():
        m_old = stat[0]
        s_old = stat[1]
        m_new = jnp.maximum(m_old, bm)
        stat[0] = m_new
        stat[1] = s_old * jnp.exp(m_old - m_new) + bs * jnp.exp(bm - m_new)

    @pl.when(i == pl.num_programs(0) - 1)
    def _():
        mp_o[...] = jnp.reshape(stat[0] + jnp.log(stat[1]), (1, 1))


def _k3(gu, gv, ef, WC, cvec, wtil, c0, kap):
    grid = E // EB
    return pl.pallas_call(
        _k3_body,
        grid=(grid,),
        in_specs=[
            pl.BlockSpec((EB, D), lambda i: (i, 0)),
            pl.BlockSpec((EB, D), lambda i: (i, 0)),
            pl.BlockSpec((EB, 16), lambda i: (i, 0)),
            pl.BlockSpec((16, D), lambda i: (0, 0)),
            pl.BlockSpec((1, D), lambda i: (0, 0)),
            pl.BlockSpec((1, D), lambda i: (0, 0)),
            pl.BlockSpec((1, 1), lambda i: (0, 0), memory_space=pltpu.SMEM),
            pl.BlockSpec((1, D), lambda i: (0, 0)),
        ],
        out_specs=[
            pl.BlockSpec((EB, D), lambda i: (i, 0)),
            pl.BlockSpec((EB, 16), lambda i: (i, 0)),
            pl.BlockSpec((1, 1), lambda i: (0, 0)),
        ],
        out_shape=[
            jax.ShapeDtypeStruct((E, D), _f32),
            jax.ShapeDtypeStruct((E, 16), _f32),
            jax.ShapeDtypeStruct((1, 1), _f32),
        ],
        scratch_shapes=[pltpu.SMEM((2,), _f32)],
        compiler_params=pltpu.CompilerParams(
            dimension_semantics=("arbitrary",)),
    )(gu, gv, ef, WC, cvec, wtil, c0, kap)


def _k5_body_full(x, agg2, W2, gl, bl, P, Q, x_o, u_o, v_o):
    s = agg2[0] + agg2[1]
    aggf = jnp.dot(s, W2[...], preferred_element_type=_f32)
    xn = jnp.maximum(_ln(x[...] + aggf, gl[...], bl[...]), 0.0)
    x_o[...] = xn
    u_o[...] = jnp.dot(xn, P[...], preferred_element_type=_f32)
    v_o[...] = jnp.dot(xn, Q[...], preferred_element_type=_f32)


def _k5_body_last(x, agg2, W2, gl, bl, x_o):
    s = agg2[0] + agg2[1]
    aggf = jnp.dot(s, W2[...], preferred_element_type=_f32)
    x_o[...] = jnp.maximum(_ln(x[...] + aggf, gl[...], bl[...]), 0.0)


def _k5(x, agg2, W2, gl, bl, P=None, Q=None):
    grid = N // RB
    last = P is None
    in_specs = [
        pl.BlockSpec((RB, D), lambda i: (i, 0)),
        pl.BlockSpec((2, RB, D), lambda i: (0, i, 0)),
        pl.BlockSpec((D, D), lambda i: (0, 0)),
        pl.BlockSpec((1, D), lambda i: (0, 0)),
        pl.BlockSpec((1, D), lambda i: (0, 0)),
    ]
    args = [x, agg2, W2, gl, bl]
    if last:
        body = _k5_body_last
        out_specs = pl.BlockSpec((RB, D), lambda i: (i, 0))
        out_shape = jax.ShapeDtypeStruct((N, D), _f32)
    else:
        body = _k5_body_full
        in_specs += [pl.BlockSpec((D, D), lambda i: (0, 0))] * 2
        args += [P, Q]
        out_specs = [pl.BlockSpec((RB, D), lambda i: (i, 0))] * 3
        out_shape = [jax.ShapeDtypeStruct((N, D), _f32)] * 3
    return pl.pallas_call(
        body,
        grid=(grid,),
        in_specs=in_specs,
        out_specs=out_specs,
        out_shape=out_shape,
        compiler_params=pltpu.CompilerParams(
            dimension_semantics=("arbitrary",)),
    )(*args)


def _k6_body(x, batch3, Wout, bo, gn, bn2, h_o, sums, cnt):
    i = pl.program_id(0)

    @pl.when(i == 0)
    def _():
        sums[...] = jnp.zeros((NG, D), _f32)
        cnt[...] = jnp.zeros((NG, D), _f32)

    ids = batch3[0, 0, :]
    iota = lax.broadcasted_iota(jnp.int32, (RB, NG), 1)
    oh = (ids[:, None] == iota).astype(_f32)
    sums[...] += lax.dot_general(oh, x[...], (((0,), (0,)), ((), ())),
                                 preferred_element_type=_f32)
    cnt[...] += jnp.broadcast_to(jnp.sum(oh, axis=0)[:, None], (NG, D))

    @pl.when(i == pl.num_programs(0) - 1)
    def _():
        pooled = sums[...] / jnp.maximum(cnt[...], 1.0)
        o = jnp.dot(pooled, Wout[...], preferred_element_type=_f32) + bo[...]
        h_o[...] = _ln(o, gn[...], bn2[...])


def _k6(x, batch3, Wout, bo, gn, bn2):
    grid = N // RB
    return pl.pallas_call(
        _k6_body,
        grid=(grid,),
        in_specs=[
            pl.BlockSpec((RB, D), lambda i: (i, 0)),
            pl.BlockSpec((1, 1, RB), lambda i: (i, 0, 0)),
            pl.BlockSpec((D, D), lambda i: (0, 0)),
            pl.BlockSpec((1, D), lambda i: (0, 0)),
            pl.BlockSpec((1, D), lambda i: (0, 0)),
            pl.BlockSpec((1, D), lambda i: (0, 0)),
        ],
        out_specs=pl.BlockSpec((NG, D), lambda i: (0, 0)),
        out_shape=jax.ShapeDtypeStruct((NG, D), _f32),
        scratch_shapes=[pltpu.VMEM((NG, D), _f32), pltpu.VMEM((NG, D), _f32)],
        compiler_params=pltpu.CompilerParams(
            dimension_semantics=("arbitrary",)),
    )(x, batch3, Wout, bo, gn, bn2)


def _k7_body(ha, hb, lw, W1d, W1a, W1b, rb1, rg, rbeta, RW2, rb2,
             HW1, hb1, HW2, hb2, out_o):
    a = ha[...]
    b = hb[...]
    diff = b - a
    w = jnp.exp(lw[...])
    dist = jnp.sqrt(jnp.sum(diff * diff * w, axis=1, keepdims=True) + 1e-8)
    r = (dist * W1d[...]
         + jnp.dot(a, W1a[...], preferred_element_type=_f32)
         + jnp.dot(b, W1b[...], preferred_element_type=_f32) + rb1[...])
    r = jnp.maximum(r, 0.0)
    r = _ln(r, rg[...], rbeta[...])
    r = jnp.maximum(jnp.dot(r, RW2[...], preferred_element_type=_f32)
                    + rb2[...], 0.0)
    h = jnp.maximum(jnp.dot(r, HW1[...], preferred_element_type=_f32)
                    + hb1[...], 0.0)
    out_o[...] = jnp.sum(h * HW2[...], axis=1, keepdims=True) + hb2[...]


def _k7(ha, hb, lw, W1d, W1a, W1b, rb1, rg, rbeta, RW2, rb2, HW1, hb1, HW2, hb2):
    full = lambda s: pl.BlockSpec(s, lambda: tuple(0 for _ in s))
    args = [ha, hb, lw, W1d, W1a, W1b, rb1, rg, rbeta, RW2, rb2, HW1, hb1, HW2, hb2]
    return pl.pallas_call(
        _k7_body,
        in_specs=[full(x.shape) for x in args],
        out_specs=full((NG, 1)),
        out_shape=jax.ShapeDtypeStruct((NG, 1), _f32),
    )(*args)


NC = 2
NS = 16
NW = NC * NS
EPW = E // NW
CH = 200
NCH = EPW // CH
CS = 160
NCS = EPW // CS
NP = 10240
RPS = NP // NS

_sc_mesh = plsc.VectorSubcoreMesh(core_axis_name="c", subcore_axis_name="s")


@functools.partial(
    pl.kernel,
    out_type=[jax.ShapeDtypeStruct((E, D), _f32)] * 2,
    mesh=_sc_mesh,
    scratch_types=[
        pltpu.VMEM((CH,), jnp.int32),
        pltpu.VMEM((CH,), jnp.int32),
        pltpu.VMEM((CH,), jnp.int32),
        pltpu.VMEM((CH,), jnp.int32),
        pltpu.VMEM((CH, D), _f32),
        pltpu.VMEM((CH, D), _f32),
        pltpu.VMEM((CH, D), _f32),
        pltpu.VMEM((CH, D), _f32),
        pltpu.SemaphoreType.DMA,
        pltpu.SemaphoreType.DMA,
        pltpu.SemaphoreType.DMA,
        pltpu.SemaphoreType.DMA,
    ],
)
def _k2_sc(u_hbm, v_hbm, dst_hbm, src_hbm, gu_hbm, gv_hbm,
           idxd0, idxd1, idxs0, idxs1, bu0, bu1, bv0, bv1,
           semi, semg, semw0, semw1):
    wid = lax.axis_index("s") * NC + lax.axis_index("c")
    base = wid * EPW
    idxd = [idxd0, idxd1]
    idxs = [idxs0, idxs1]
    bu = [bu0, bu1]
    bv = [bv0, bv1]
    semw = [semw0, semw1]

    pend_idx = [None] * NCH
    pend_w = [None] * NCH
    pltpu.sync_copy(dst_hbm.at[pl.ds(base, CH)], idxd[0])
    pltpu.sync_copy(src_hbm.at[pl.ds(base, CH)], idxs[0])
    for i in range(NCH):
        b = i % 2
        off = base + i * CH
        if i + 1 < NCH:
            noff = off + CH
            pend_idx[i + 1] = (
                pltpu.async_copy(dst_hbm.at[pl.ds(noff, CH)], idxd[1 - b],
                                 semi),
                pltpu.async_copy(src_hbm.at[pl.ds(noff, CH)], idxs[1 - b],
                                 semi),
            )
        if i >= 2:
            for c in pend_w[i - 2]:
                c.wait()
        cu = pltpu.async_copy(u_hbm.at[idxd[b]], bu[b], semg)
        cv = pltpu.async_copy(v_hbm.at[idxs[b]], bv[b], semg)
        cu.wait()
        cv.wait()
        pend_w[i] = (
            pltpu.async_copy(bu[b], gu_hbm.at[pl.ds(off, CH)], semw[b]),
            pltpu.async_copy(bv[b], gv_hbm.at[pl.ds(off, CH)], semw[b]),
        )
        if i + 1 < NCH:
            for c in pend_idx[i + 1]:
                c.wait()
    for c in pend_w[NCH - 2] + pend_w[NCH - 1]:
        c.wait()


@functools.partial(
    pl.kernel,
    out_type=jax.ShapeDtypeStruct((NC, NP, D), _f32),
    mesh=_sc_mesh,
    scratch_types=[
        pltpu.VMEM_SHARED((NP, D), _f32),
        pltpu.VMEM((CS,), jnp.int32),
        pltpu.VMEM((EPW - (EPW // CS) * CS,), jnp.int32),
        pltpu.VMEM((CS, D), _f32),
        pltpu.VMEM((CS, 16), _f32),
        pltpu.VMEM((16,), _f32),
    ],
)
def _k4_sc(m_hbm, arep_hbm, mp_hbm, dst_hbm, agg_hbm,
           shared, idx, idx2, bm, ba, mp_v):
    cid = lax.axis_index("c")
    sid = lax.axis_index("s")
    base = cid * (E // NC) + sid * EPW

    pltpu.sync_copy(mp_hbm, mp_v)

    zeros16 = jnp.zeros((16,), _f32)

    def zrow(r, carry):
        for k in range(D // 16):
            bm[r, pl.ds(k * 16, 16)] = zeros16
        return carry

    lax.fori_loop(0, CS, zrow, 0)
    for o in range(0, RPS, CS):
        sz = min(CS, RPS - o)
        pltpu.sync_copy(bm.at[pl.ds(0, sz)],
                        shared.at[pl.ds(sid * RPS + o, sz)])
    plsc.subcore_barrier()

    def do_chunk(off, sz, idxr):
        pltpu.sync_copy(dst_hbm.at[pl.ds(off, sz)], idxr)
        bms = bm if sz == CS else bm.at[pl.ds(0, sz)]
        pltpu.sync_copy(m_hbm.at[pl.ds(off, sz)], bms)
        pltpu.sync_copy(arep_hbm.at[pl.ds(off, sz)], ba.at[pl.ds(0, sz)])

        def row(r, c2):
            w = jnp.exp(ba[r, pl.ds(0, 16)] - mp_v[...])
            for k in range(D // 16):
                sl = pl.ds(k * 16, 16)
                bm[r, sl] = bm[r, sl] * w
            return c2

        lax.fori_loop(0, sz, row, 0)
        pltpu.sync_copy(bms, shared.at[idxr], add=True)

    def chunk(i, carry):
        do_chunk(base + i * CS, CS, idx)
        return carry

    lax.fori_loop(0, NCS, chunk, 0)
    do_chunk(base + NCS * CS, EPW - NCS * CS, idx2)
    plsc.subcore_barrier()

    pltpu.sync_copy(shared.at[pl.ds(sid * RPS, RPS)],
                    agg_hbm.at[cid, pl.ds(sid * RPS, RPS)])


def _gather(u, v, src, dst):
    return _k2_sc(u, v, dst, src)


def _scatter(m, arep, mp, dst):
    mp16 = jnp.broadcast_to(mp.reshape(1), (16,))
    return _k4_sc(m, arep, mp16, dst)


def _encode(nf, ei, ef, batch, Wn, bn, We, be, W1s, b1s, W2s, b2s, Was, bas,
            gs, betas, Wout, bo, gn, bn2):
    src = ei[0]
    dst = ei[1]
    Ps, Qs, WCs, cvecs, wtils, c0s, kappas = [], [], [], [], [], [], []
    for l in range(3):
        A, B, C = W1s[l][:D], W1s[l][D:2 * D], W1s[l][2 * D:]
        Ps.append(A - B)
        Qs.append(B)
        WCs.append(We @ C)
        cvecs.append((be @ C + b1s[l]).reshape(1, D))
        wt = (W2s[l] @ Was[l]).reshape(1, D)
        wtils.append(wt)
        c0s.append((b2s[l] @ Was[l] + bas[l]).reshape(1, 1))
        kappas.append(jnp.linalg.solve(W2s[l].T, b2s[l]).reshape(1, D))

    x, u, v = _k1(nf, Wn, bn.reshape(1, D), Ps[0], Qs[0])
    for l in range(3):
        gu, gv = _gather(u, v, src, dst)
        m, arep, mp = _k3(gu, gv, ef, WCs[l], cvecs[l], wtils[l], c0s[l],
                          kappas[l])
        agg2 = _scatter(m, arep, mp, dst)
        if l < 2:
            x, u, v = _k5(x, agg2, W2s[l],
                          gs[l].reshape(1, D), betas[l].reshape(1, D),
                          Ps[l + 1], Qs[l + 1])
        else:
            x = _k5(x, agg2, W2s[l],
                    gs[l].reshape(1, D), betas[l].reshape(1, D))
    batch3 = batch.reshape(N // RB, 1, RB)
    return _k6(x, batch3, Wout, bo.reshape(1, D), gn.reshape(1, D),
               bn2.reshape(1, D))


def kernel(node_embed_W, node_embed_b, edge_embed_W, edge_embed_b, conv_W1,
           conv_b1, conv_W2, conv_b2, conv_Wa, conv_ba, conv_g, conv_beta,
           out_proj_W, out_proj_b, out_norm_g, out_norm_b, log_weights,
           res_W1, res_b1, res_g, res_beta, res_W2, res_b2, head_W1, head_b1,
           head_W2, head_b2, node_features_a, edge_index_a, edge_features_a,
           batch_a, node_features_b, edge_index_b, edge_features_b, batch_b):
    enc = functools.partial(
        _encode, Wn=node_embed_W, bn=node_embed_b, We=edge_embed_W,
        be=edge_embed_b, W1s=conv_W1, b1s=conv_b1, W2s=conv_W2, b2s=conv_b2,
        Was=conv_Wa, bas=conv_ba, gs=conv_g, betas=conv_beta, Wout=out_proj_W,
        bo=out_proj_b, gn=out_norm_g, bn2=out_norm_b)
    h_a = enc(node_features_a, edge_index_a, edge_features_a, batch_a)
    h_b = enc(node_features_b, edge_index_b, edge_features_b, batch_b)
    return _k7(h_a, h_b, log_weights.reshape(1, D),
               res_W1[0:1], res_W1[1:D + 1], res_W1[D + 1:2 * D + 1],
               res_b1.reshape(1, D), res_g.reshape(1, D),
               res_beta.reshape(1, D), res_W2, res_b2.reshape(1, D // 2),
               head_W1, head_b1.reshape(1, 32), head_W2.reshape(1, 32),
               head_b2.reshape(1, 1))
